# baseline (speedup 1.0000x reference)
import jax
import jax.numpy as jnp
from jax import lax
from jax.experimental import pallas as pl
from jax.experimental.pallas import tpu as pltpu
from jax.experimental.pallas import tpu_sc as plsc

NC = 2
NS = 16
LN = 16
CH = 64
EW = 2 * CH
GRP = 16
NB = 4
ZB = 128

_BCAST_DNUMS = lax.GatherDimensionNumbers(
    offset_dims=(), collapsed_slice_dims=(0,), start_index_map=(0,))


def _bcast_lane(vec, e):
    idx = jnp.full((LN, 1), e, jnp.int32)
    return lax.gather(vec, idx, _BCAST_DNUMS, (1,),
                      mode=lax.GatherScatterMode.PROMISE_IN_BOUNDS)


def _zero_buf(ref, nrows, d):
    z = jnp.zeros((LN,), jnp.float32)

    def body(r, carry):
        for k in range(d // LN):
            ref[r, pl.ds(k * LN, LN)] = z
        return carry

    lax.fori_loop(0, nrows, body, 0)


def _scale_chunk(gbig, bofs, edv, edvf, erow, idx2, bi, d,
                 tile_lo=None, tile_rows=None):
    def grp_fn(g, carry):
        rv = edv[erow, pl.ds(CH + g * LN, LN)]
        vv = edvf[erow, pl.ds(g * LN, LN)]
        if tile_lo is not None:
            ok = (rv >= tile_lo) & (rv < tile_lo + tile_rows)
            vv = jnp.where(ok, vv, 0.0)
            rv = jnp.clip(rv - tile_lo, 0, tile_rows - 1)
        idx2[bi, pl.ds(g * LN, LN)] = rv
        for e in range(LN):
            val = _bcast_lane(vv, e)
            r = bofs + g * LN + e
            for k in range(d // LN):
                gbig[r, pl.ds(k * LN, LN)] = gbig[r, pl.ds(k * LN, LN)] * val
        return carry

    lax.fori_loop(0, CH // LN, grp_fn, 0)


def _row_partition(n):
    rps8 = -(-(-(-n // NS)) // 8) * 8
    last = n - (NS - 1) * rps8
    assert last > 0 and last % 8 == 0 and n % 8 == 0
    return rps8, last


def _zero_acc(s, gbig, acc, rps8, last):
    for cnt, pred in ((rps8, s < NS - 1), (last, s == NS - 1)):
        nfull, nrem = divmod(cnt, ZB)

        @pl.when(pred)
        def _():
            base = s * rps8
            for j in range(nfull):
                pltpu.sync_copy(gbig.at[pl.ds(0, ZB)],
                                acc.at[pl.ds(base + j * ZB, ZB)])
            if nrem:
                pltpu.sync_copy(gbig.at[pl.ds(0, nrem)],
                                acc.at[pl.ds(base + nfull * ZB, nrem)])


def _writeout(s, acc, out, out_off, rps8, last):
    for cnt, pred in ((rps8, s < NS - 1), (last, s == NS - 1)):

        @pl.when(pred)
        def _():
            pltpu.sync_copy(acc.at[pl.ds(s * rps8, cnt)],
                            out.at[pl.ds(out_off + s * rps8, cnt)])


def _meta_copies(edata, edataf, edv, edvf, row0, half, sem):
    a = pltpu.make_async_copy(edata.at[pl.ds(row0, GRP)],
                              edv.at[pl.ds(half * GRP, GRP)], sem)
    b = pltpu.make_async_copy(edataf.at[pl.ds(row0, GRP)],
                              edvf.at[pl.ds(half * GRP, GRP)], sem)
    return a, b


def _spmm_pipeline(table, edata, edataf, acc, edv, edvf, gbig, idx2,
                   esems, gsems, ssems, base_row, mc, ng, d,
                   tile_lo=None, tile_rows=None):
    def fire_gather(erow, b):
        pltpu.async_copy(table.at[edv.at[erow, pl.ds(0, CH)]],
                         gbig.at[pl.ds(b * CH, CH)], gsems[b])

    def wait_gather(b):
        pltpu.make_async_copy(table.at[edv.at[0, pl.ds(0, CH)]],
                              gbig.at[pl.ds(b * CH, CH)], gsems[b]).wait()

    def fire_scatter(b):
        pltpu.async_copy(gbig.at[pl.ds(b * CH, CH)], acc.at[idx2.at[b]],
                         ssems[b], add=True)

    def wait_scatter(b):
        pltpu.make_async_copy(gbig.at[pl.ds(b * CH, CH)],
                              acc.at[idx2.at[b]], ssems[b]).wait()

    @pl.when(ng > 0)
    def _():
        for cp in _meta_copies(edata, edataf, edv, edvf, base_row, 0,
                               esems[0]):
            cp.start()
        for cp in _meta_copies(edata, edataf, edv, edvf, base_row, 0,
                               esems[0]):
            cp.wait()

    for b in range(2):
        @pl.when(mc > b)
        def _():
            fire_gather(b, b)

    def it(k, carry):
        g = k // GRP
        rk = lax.rem(k, GRP)

        @pl.when((rk == 0) & (g + 1 < ng))
        def _():
            for h in range(2):
                @pl.when(lax.rem(g + 1, 2) == h)
                def _():
                    for cp in _meta_copies(edata, edataf, edv, edvf,
                                           base_row + (g + 1) * GRP, h,
                                           esems[h]):
                        cp.start()

        @pl.when((rk == GRP - 2) & (g + 1 < ng))
        def _():
            for h in range(2):
                @pl.when(lax.rem(g + 1, 2) == h)
                def _():
                    for cp in _meta_copies(edata, edataf, edv, edvf,
                                           base_row + (g + 1) * GRP, h,
                                           esems[h]):
                        cp.wait()

        bi = lax.rem(k, NB)
        erow = lax.rem(k, 2 * GRP)
        for b in range(NB):
            @pl.when(bi == b)
            def _():
                wait_gather(b)
        _scale_chunk(gbig, bi * CH, edv, edvf, erow, idx2, bi, d,
                     tile_lo, tile_rows)
        for b in range(NB):
            @pl.when(bi == b)
            def _():
                fire_scatter(b)

        @pl.when(k + 2 < mc)
        def _():
            b2v = lax.rem(k + 2, NB)
            er2 = lax.rem(k + 2, 2 * GRP)
            for b in range(NB):
                @pl.when(b2v == b)
                def _():
                    @pl.when(k >= 2)
                    def _():
                        wait_scatter(b)
                    fire_gather(er2, b)
        return carry

    lax.fori_loop(0, mc, it, 0)

    def drain(k, carry):
        for b in range(NB):
            @pl.when(lax.rem(k, NB) == b)
            def _():
                wait_scatter(b)
        return carry

    lax.fori_loop(jnp.maximum(mc - 4, 0), mc, drain, 0)


def _phase_a_body(G, D, nch):
    rps8, last = _row_partition(G)
    assert nch % GRP == 0

    def body(table, edata, edataf, out, edv, edvf, gbig, idx2, acc,
             se0, se1, sg0, sg1, sg2, sg3, ss0, ss1, ss2, ss3):
        c = lax.axis_index("c")
        s = lax.axis_index("s")
        wid = c * NS + s
        _zero_buf(gbig, ZB, D)
        _zero_acc(s, gbig, acc, rps8, last)
        plsc.subcore_barrier()
        _spmm_pipeline(table, edata, edataf, acc, edv, edvf, gbig, idx2,
                       (se0, se1), (sg0, sg1, sg2, sg3),
                       (ss0, ss1, ss2, ss3), wid * nch, nch, nch // GRP, D)
        plsc.subcore_barrier()
        _writeout(s, acc, out, c * G, rps8, last)

    return body


def _phase_c_body(G, D, n_out, tile_rows):
    n_tiles = n_out // tile_rows
    tpc = n_tiles // NC
    rps8, last = _row_partition(tile_rows)

    def body(msg, edata, edataf, ptr, out, edv, edvf, gbig, idx2, ptrb, acc,
             se0, se1, sg0, sg1, sg2, sg3, ss0, ss1, ss2, ss3):
        c = lax.axis_index("c")
        s = lax.axis_index("s")
        pltpu.sync_copy(ptr, ptrb)
        for tl in range(tpc):
            t = c * tpc + tl
            pv = ptrb[pl.ds(t, LN)]
            lo = pv[0]
            hi = pv[1]
            j0 = (lo // (8 * CH)) * 8
            total_ch = (hi - j0 * CH + CH - 1) // CH
            bc = ((total_ch + NS * 8 - 1) // (NS * 8)) * 8
            mc = jnp.clip(total_ch - s * bc, 0, bc)
            ng = (mc + GRP - 1) // GRP
            tile_lo = t * tile_rows
            _zero_buf(gbig, ZB, D)
            _zero_acc(s, gbig, acc, rps8, last)
            plsc.subcore_barrier()
            _spmm_pipeline(msg, edata, edataf, acc, edv, edvf, gbig, idx2,
                           (se0, se1), (sg0, sg1, sg2, sg3),
                           (ss0, ss1, ss2, ss3), j0 + s * bc, mc, ng, D,
                           tile_lo, tile_rows)
            plsc.subcore_barrier()
            _writeout(s, acc, out, tile_lo, rps8, last)
            plsc.subcore_barrier()

    return body


def _padto(x, n, fill):
    m = x.shape[0]
    if m == n:
        return x
    return jnp.concatenate([x, jnp.full((n - m,), fill, x.dtype)])


def _pack_edges(cols, rows, vals, n_edges_pad):
    c = _padto(cols, n_edges_pad, 0).reshape(-1, CH)
    r = _padto(rows, n_edges_pad, 0).reshape(-1, CH)
    v = _padto(vals, n_edges_pad, 0.0).reshape(-1, CH)
    return jnp.concatenate([c, r], axis=1), v


def kernel(user_emb, item_emb, group_emb, uh_rows, uh_cols, uh_vals,
           ih_rows, ih_cols, ih_vals, fh_rows, fh_cols, fh_vals,
           W0, b0, W1, b1, num_users, num_items):
    f32 = jnp.float32
    U, D = user_emb.shape
    I = item_emb.shape[0]
    G = group_emb.shape[0]
    N = U + I

    emb0 = jnp.concatenate([user_emb, item_emb], axis=0)

    nnz_a = max(uh_rows.shape[0], ih_rows.shape[0])
    nch_a = -(-(-(-nnz_a // NS)) // CH)
    nch_a = -(-nch_a // GRP) * GRP
    apad = NS * nch_a * CH
    ua_i, ua_v = _pack_edges(uh_cols, uh_rows, uh_vals, apad)
    ia_i, ia_v = _pack_edges(ih_cols + U, ih_rows, ih_vals, apad)
    edata_a = jnp.concatenate([ua_i, ia_i])
    edataf_a = jnp.concatenate([ua_v, ia_v])

    nnz_f = fh_rows.shape[0]
    TILE = 10000
    n_tiles = N // TILE
    nch_f = -(-nnz_f // CH)
    bc_max = ((nch_f + 7 + NS * 8 - 1) // (NS * 8)) * 8
    fpad = (nch_f + bc_max + 2 * GRP) * CH
    edata_f, edataf_f = _pack_edges(fh_cols, fh_rows, fh_vals, fpad)
    ptr = jnp.searchsorted(
        fh_rows,
        jnp.arange(n_tiles + 1, dtype=jnp.int32) * TILE).astype(jnp.int32)
    ptr32 = _padto(ptr, 2 * LN, nnz_f)

    mesh = plsc.VectorSubcoreMesh(core_axis_name="c", subcore_axis_name="s",
                                  num_cores=NC, num_subcores=NS)
    sems = [pltpu.SemaphoreType.DMA] * 10

    phase_a = pl.kernel(
        _phase_a_body(G, D, nch_a),
        out_type=jax.ShapeDtypeStruct((NC * G, D), f32),
        mesh=mesh,
        scratch_types=[
            pltpu.VMEM((2 * GRP, EW), jnp.int32),
            pltpu.VMEM((2 * GRP, CH), f32),
            pltpu.VMEM((NB * CH, D), f32),
            pltpu.VMEM((NB, CH), jnp.int32),
            pltpu.VMEM_SHARED((G, D), f32),
        ] + sems,
    )

    phase_c = pl.kernel(
        _phase_c_body(G, D, N, TILE),
        out_type=jax.ShapeDtypeStruct((N, D), f32),
        mesh=mesh,
        scratch_types=[
            pltpu.VMEM((2 * GRP, EW), jnp.int32),
            pltpu.VMEM((2 * GRP, CH), f32),
            pltpu.VMEM((NB * CH, D), f32),
            pltpu.VMEM((NB, CH), jnp.int32),
            pltpu.VMEM((2 * LN,), jnp.int32),
            pltpu.VMEM_SHARED((TILE, D), f32),
        ] + sems,
    )

    BLK = 2000

    def _combine(um_ref, im_ref, g_ref, W_ref, b_ref, he_ref, msg_ref, heo_ref):
        um = um_ref[...]
        im = im_ref[...]
        gg = g_ref[...]
        W = W_ref[...]
        m = (jnp.dot(um, W[0:D], preferred_element_type=f32)
             + jnp.dot(im, W[D:2 * D], preferred_element_type=f32)
             + jnp.dot(im * gg, W[2 * D:3 * D], preferred_element_type=f32)
             + b_ref[...])
        msg_ref[...] = m
        heo_ref[...] = he_ref[...] + m

    combine = pl.pallas_call(
        _combine,
        grid=(G // BLK,),
        in_specs=[pl.BlockSpec((BLK, D), lambda i: (i, 0))] * 3
        + [pl.BlockSpec((3 * D, D), lambda i: (0, 0)),
           pl.BlockSpec((1, D), lambda i: (0, 0)),
           pl.BlockSpec((BLK, D), lambda i: (i, 0))],
        out_specs=[pl.BlockSpec((BLK, D), lambda i: (i, 0))] * 2,
        out_shape=[jax.ShapeDtypeStruct((G, D), f32)] * 2,
    )

    BLK3 = 4000

    def _sum3(a_ref, b_ref, c_ref, o_ref):
        o_ref[...] = a_ref[...] + b_ref[...] + c_ref[...]

    sum3 = pl.pallas_call(
        _sum3,
        grid=(N // BLK3,),
        in_specs=[pl.BlockSpec((BLK3, D), lambda i: (i, 0))] * 3,
        out_specs=pl.BlockSpec((BLK3, D), lambda i: (i, 0)),
        out_shape=jax.ShapeDtypeStruct((N, D), f32),
    )

    b0r = b0.reshape(1, D)
    b1r = b1.reshape(1, D)

    msgs1 = phase_a(emb0, edata_a, edataf_a)
    msg1, he1 = combine(msgs1[:G], msgs1[G:], group_emb, W0, b0r, group_emb)
    emb1 = phase_c(msg1, edata_f, edataf_f, ptr32)
    msgs2 = phase_a(emb1, edata_a, edataf_a)
    msg2, he2 = combine(msgs2[:G], msgs2[G:], group_emb, W1, b1r, he1)
    emb2 = phase_c(msg2, edata_f, edataf_f, ptr32)

    final_emb = sum3(emb0, emb1, emb2)
    return (final_emb, he2)
